# baseline (speedup 1.0000x reference)
import functools
import numpy as np
import jax, jax.numpy as jnp
from jax import lax
from jax.experimental import pallas as pl
from jax.experimental.pallas import tpu as pltpu
from jax.experimental.pallas import tpu_sc as plsc

S, D, H, E, K = 2048, 1024, 2048, 8, 2
BLK = 256
NB = (S * K) // BLK + E
NP = NB * BLK


def _gating_body(x_ref, wg_ref, bg_ref, logits_ref, idx_ref, w_ref):
    x = x_ref[...]
    lg = jax.lax.dot_general(x, wg_ref[...], (((1,), (0,)), ((), ())),
                             preferred_element_type=jnp.float32,
                             precision=jax.lax.Precision.DEFAULT)
    lg = lg + bg_ref[...]
    logits_ref[...] = lg
    ii = jax.lax.broadcasted_iota(jnp.int32, lg.shape, 1)
    m0 = jnp.max(lg, axis=1, keepdims=True)
    i0 = jnp.min(jnp.where(lg == m0, ii, E), axis=1, keepdims=True)
    lg2 = jnp.where(ii == i0, -jnp.inf, lg)
    m1 = jnp.max(lg2, axis=1, keepdims=True)
    i1 = jnp.min(jnp.where(lg2 == m1, ii, E), axis=1, keepdims=True)
    t = jnp.exp(m1 - m0)
    w0 = 1.0 / (1.0 + t)
    w1 = t / (1.0 + t)
    idx_ref[...] = jnp.concatenate([i0, i1], axis=1)
    w_ref[...] = jnp.concatenate([w0, w1], axis=1)


def _gating(x_flat, Wg, bg):
    return pl.pallas_call(
        _gating_body,
        out_shape=(
            jax.ShapeDtypeStruct((S, E), jnp.float32),
            jax.ShapeDtypeStruct((S, K), jnp.int32),
            jax.ShapeDtypeStruct((S, K), jnp.float32),
        ),
    )(x_flat, Wg, bg.reshape(1, E))


def _ffn_body(be_ref, nact_ref, xs_ref, w1_ref, b1_ref, w2_ref, b2_ref, wt_ref,
              out_ref):
    @pl.when(pl.program_id(0) < nact_ref[0])
    def _():
        h = jax.lax.dot_general(xs_ref[...], w1_ref[0], (((1,), (0,)), ((), ())),
                                preferred_element_type=jnp.float32,
                                precision=jax.lax.Precision.DEFAULT)
        h = h + b1_ref[0]
        h = 0.5 * h * (1.0 + jax.lax.erf(h / np.sqrt(2).astype(np.float32)))
        o = jax.lax.dot_general(h, w2_ref[0], (((1,), (0,)), ((), ())),
                                preferred_element_type=jnp.float32,
                                precision=jax.lax.Precision.DEFAULT)
        out_ref[...] = (o + b2_ref[0]) * wt_ref[...]


def _ffn(xs, W1, b1, W2, b2, sorted_wt, blk_expert, nact):
    grid_spec = pltpu.PrefetchScalarGridSpec(
        num_scalar_prefetch=2,
        grid=(NB,),
        in_specs=[
            pl.BlockSpec((BLK, D), lambda i, be, na: (i, 0)),
            pl.BlockSpec((1, D, H), lambda i, be, na: (be[i], 0, 0)),
            pl.BlockSpec((1, 1, H), lambda i, be, na: (be[i], 0, 0)),
            pl.BlockSpec((1, H, D), lambda i, be, na: (be[i], 0, 0)),
            pl.BlockSpec((1, 1, D), lambda i, be, na: (be[i], 0, 0)),
            pl.BlockSpec((BLK, 1), lambda i, be, na: (i, 0)),
        ],
        out_specs=pl.BlockSpec((BLK, D), lambda i, be, na: (i, 0)),
    )
    return pl.pallas_call(
        _ffn_body,
        grid_spec=grid_spec,
        out_shape=jax.ShapeDtypeStruct((NP, D), jnp.float32),
        compiler_params=pltpu.CompilerParams(
            dimension_semantics=("arbitrary",)),
    )(blk_expert, nact, xs, W1, b1.reshape(E, 1, H), W2, b2.reshape(E, 1, D),
      sorted_wt.reshape(NP, 1))


NW = 32
GCHUNK = 64


def _sc_gather(x_flat, sorted_token):
    per_w = NP // NW
    nchunk = per_w // GCHUNK
    mesh = plsc.VectorSubcoreMesh(core_axis_name="c", subcore_axis_name="s")

    @functools.partial(
        pl.kernel, mesh=mesh,
        out_type=jax.ShapeDtypeStruct((NP, D), jnp.float32),
        scratch_types=[
            pltpu.VMEM((GCHUNK,), jnp.int32),
            pltpu.VMEM((GCHUNK, D), jnp.float32),
            pltpu.SemaphoreType.DMA,
        ],
    )
    def k(x_hbm, idx_hbm, out_hbm, idx_v, rows_v, sem):
        wid = lax.axis_index("s") * 2 + lax.axis_index("c")
        for c in range(nchunk):
            base = wid * per_w + c * GCHUNK
            pltpu.sync_copy(idx_hbm.at[pl.ds(base, GCHUNK)], idx_v)
            pltpu.async_copy(x_hbm.at[idx_v], rows_v, sem).wait()
            pltpu.sync_copy(rows_v, out_hbm.at[pl.ds(base, GCHUNK)])

    return k(x_flat, sorted_token)


CCHUNK = 32


def _sc_combine(ys, pos0, pos1):
    per_w = S // NW
    nchunk = per_w // CCHUNK
    mesh = plsc.VectorSubcoreMesh(core_axis_name="c", subcore_axis_name="s")

    @functools.partial(
        pl.kernel, mesh=mesh,
        out_type=jax.ShapeDtypeStruct((S, D), jnp.float32),
        scratch_types=[
            pltpu.VMEM((CCHUNK,), jnp.int32),
            pltpu.VMEM((CCHUNK,), jnp.int32),
            pltpu.VMEM((CCHUNK, D), jnp.float32),
            pltpu.VMEM((CCHUNK, D), jnp.float32),
            pltpu.VMEM((CCHUNK, D), jnp.float32),
            pltpu.SemaphoreType.DMA,
            pltpu.SemaphoreType.DMA,
        ],
    )
    def k(ys_hbm, p0_hbm, p1_hbm, out_hbm,
          i0_v, i1_v, g0_v, g1_v, o_v, sem0, sem1):
        wid = lax.axis_index("s") * 2 + lax.axis_index("c")
        for c in range(nchunk):
            base = wid * per_w + c * CCHUNK
            pltpu.sync_copy(p0_hbm.at[pl.ds(base, CCHUNK)], i0_v)
            pltpu.sync_copy(p1_hbm.at[pl.ds(base, CCHUNK)], i1_v)
            cp0 = pltpu.async_copy(ys_hbm.at[i0_v], g0_v, sem0)
            cp1 = pltpu.async_copy(ys_hbm.at[i1_v], g1_v, sem1)
            cp0.wait()
            cp1.wait()

            def row(r, _):
                def col(j, _):
                    sl = pl.ds(j * 16, 16)
                    o_v[r, sl] = g0_v[r, sl] + g1_v[r, sl]
                    return 0

                return lax.fori_loop(0, D // 16, col, 0)

            lax.fori_loop(0, CCHUNK, row, 0)
            pltpu.sync_copy(o_v, out_hbm.at[pl.ds(base, CCHUNK)])

    return k(ys, pos0, pos1)


def _route(idx):
    e = idx.reshape(-1)
    oh = jax.nn.one_hot(e, E, dtype=jnp.int32)
    counts = oh.sum(axis=0)
    rank = (jnp.cumsum(oh, axis=0) - oh)[jnp.arange(S * K), e]
    blocks_per_e = (counts + BLK - 1) // BLK
    blk_start_e = jnp.cumsum(blocks_per_e) - blocks_per_e
    pos = blk_start_e[e] * BLK + rank
    nact = jnp.sum(blocks_per_e)
    cumblocks = jnp.cumsum(blocks_per_e)
    bids = jnp.arange(NB, dtype=jnp.int32)
    blk_expert = jnp.minimum(
        jnp.searchsorted(cumblocks, bids, side="right").astype(jnp.int32), E - 1)
    sorted_token = jnp.zeros((NP,), jnp.int32).at[pos].set(
        jnp.arange(S * K, dtype=jnp.int32) // K)
    return (pos.reshape(S, K), blk_expert, nact.reshape(1).astype(jnp.int32),
            sorted_token)


def kernel(x, Wg, bg, W1, b1, W2, b2):
    Bx, Sx, Dx = x.shape
    x_flat = x.reshape(-1, Dx)
    logits, idx, w = _gating(x_flat, Wg, bg)
    pos, blk_expert, nact, sorted_token = _route(idx)
    sorted_wt = jnp.zeros((NP,), jnp.float32).at[pos.reshape(-1)].set(w.reshape(-1))
    xs = _sc_gather(x_flat, sorted_token)
    ys = _ffn(xs, W1, b1, W2, b2, sorted_wt, blk_expert, nact)
    y = _sc_combine(ys, pos[:, 0], pos[:, 1])
    return (y.reshape(Bx, Sx, Dx), logits.reshape(Bx, Sx, E),
            idx.reshape(Bx, Sx, K))


# speedup vs baseline: 1.0091x; 1.0091x over previous
import functools
import numpy as np
import jax, jax.numpy as jnp
from jax import lax
from jax.experimental import pallas as pl
from jax.experimental.pallas import tpu as pltpu
from jax.experimental.pallas import tpu_sc as plsc

S, D, H, E, K = 2048, 1024, 2048, 8, 2
BLK = 256
NB = (S * K) // BLK + E
NP = NB * BLK


def _gating_body(x_ref, wg_ref, bg_ref, logits_ref, idx_ref, w_ref):
    x = x_ref[...]
    lg = jax.lax.dot_general(x, wg_ref[...], (((1,), (0,)), ((), ())),
                             preferred_element_type=jnp.float32,
                             precision=jax.lax.Precision.DEFAULT)
    lg = lg + bg_ref[...]
    logits_ref[...] = lg
    ii = jax.lax.broadcasted_iota(jnp.int32, lg.shape, 1)
    m0 = jnp.max(lg, axis=1, keepdims=True)
    i0 = jnp.min(jnp.where(lg == m0, ii, E), axis=1, keepdims=True)
    lg2 = jnp.where(ii == i0, -jnp.inf, lg)
    m1 = jnp.max(lg2, axis=1, keepdims=True)
    i1 = jnp.min(jnp.where(lg2 == m1, ii, E), axis=1, keepdims=True)
    t = jnp.exp(m1 - m0)
    w0 = 1.0 / (1.0 + t)
    w1 = t / (1.0 + t)
    idx_ref[...] = jnp.concatenate([i0, i1], axis=1)
    w_ref[...] = jnp.concatenate([w0, w1], axis=1)


def _gating(x_flat, Wg, bg):
    return pl.pallas_call(
        _gating_body,
        out_shape=(
            jax.ShapeDtypeStruct((S, E), jnp.float32),
            jax.ShapeDtypeStruct((S, K), jnp.int32),
            jax.ShapeDtypeStruct((S, K), jnp.float32),
        ),
    )(x_flat, Wg, bg.reshape(1, E))


def _ffn_body(be_ref, nact_ref, xs_ref, w1_ref, b1_ref, w2_ref, b2_ref, wt_ref,
              out_ref):
    @pl.when(pl.program_id(0) < nact_ref[0])
    def _():
        h = jax.lax.dot_general(xs_ref[...], w1_ref[0], (((1,), (0,)), ((), ())),
                                preferred_element_type=jnp.float32,
                                precision=jax.lax.Precision.DEFAULT)
        h = h + b1_ref[0]
        h = 0.5 * h * (1.0 + jax.lax.erf(h / np.sqrt(2).astype(np.float32)))
        o = jax.lax.dot_general(h, w2_ref[0], (((1,), (0,)), ((), ())),
                                preferred_element_type=jnp.float32,
                                precision=jax.lax.Precision.DEFAULT)
        out_ref[...] = (o + b2_ref[0]) * wt_ref[...]


def _ffn(xs, W1, b1, W2, b2, sorted_wt, blk_expert, nact):
    grid_spec = pltpu.PrefetchScalarGridSpec(
        num_scalar_prefetch=2,
        grid=(NB,),
        in_specs=[
            pl.BlockSpec((BLK, D), lambda i, be, na: (i, 0)),
            pl.BlockSpec((1, D, H), lambda i, be, na: (be[i], 0, 0)),
            pl.BlockSpec((1, 1, H), lambda i, be, na: (be[i], 0, 0)),
            pl.BlockSpec((1, H, D), lambda i, be, na: (be[i], 0, 0)),
            pl.BlockSpec((1, 1, D), lambda i, be, na: (be[i], 0, 0)),
            pl.BlockSpec((BLK, 1), lambda i, be, na: (i, 0)),
        ],
        out_specs=pl.BlockSpec((BLK, D), lambda i, be, na: (i, 0)),
    )
    return pl.pallas_call(
        _ffn_body,
        grid_spec=grid_spec,
        out_shape=jax.ShapeDtypeStruct((NP, D), jnp.float32),
        compiler_params=pltpu.CompilerParams(
            dimension_semantics=("arbitrary",)),
    )(blk_expert, nact, xs, W1, b1.reshape(E, 1, H), W2, b2.reshape(E, 1, D),
      sorted_wt.reshape(NP, 1))


NW = 32
GCHUNK = 32


def _sc_gather(table, idx, n_rows):
    per_w = n_rows // NW
    nchunk = per_w // GCHUNK
    mesh = plsc.VectorSubcoreMesh(core_axis_name="c", subcore_axis_name="s")

    @functools.partial(
        pl.kernel, mesh=mesh,
        out_type=jax.ShapeDtypeStruct((n_rows, D), jnp.float32),
        scratch_types=[
            pltpu.VMEM((GCHUNK,), jnp.int32),
            pltpu.VMEM((GCHUNK,), jnp.int32),
            pltpu.VMEM((GCHUNK, D), jnp.float32),
            pltpu.VMEM((GCHUNK, D), jnp.float32),
            pltpu.SemaphoreType.DMA,
            pltpu.SemaphoreType.DMA,
        ],
    )
    def k(tab_hbm, idx_hbm, out_hbm, i0_v, i1_v, r0_v, r1_v, sem0, sem1):
        wid = lax.axis_index("s") * 2 + lax.axis_index("c")
        ivs, rvs, sems = (i0_v, i1_v), (r0_v, r1_v), (sem0, sem1)

        def base(c):
            return wid * per_w + c * GCHUNK

        pltpu.sync_copy(idx_hbm.at[pl.ds(base(0), GCHUNK)], i0_v)
        cps = {0: pltpu.async_copy(tab_hbm.at[i0_v], r0_v, sem0)}
        for c in range(nchunk):
            b, nb = c % 2, (c + 1) % 2
            if c + 1 < nchunk:
                pltpu.sync_copy(idx_hbm.at[pl.ds(base(c + 1), GCHUNK)], ivs[nb])
            cps[c].wait()
            if c + 1 < nchunk:
                cps[c + 1] = pltpu.async_copy(tab_hbm.at[ivs[nb]], rvs[nb],
                                              sems[nb])
            pltpu.sync_copy(rvs[b], out_hbm.at[pl.ds(base(c), GCHUNK)])

    return k(table, idx)


def _add_body(a_ref, b_ref, y_ref):
    y_ref[...] = a_ref[...] + b_ref[...]


def _tc_add(z):
    nb = S // BLK
    return pl.pallas_call(
        _add_body,
        grid=(nb,),
        in_specs=[
            pl.BlockSpec((BLK, D), lambda i: (i, 0)),
            pl.BlockSpec((BLK, D), lambda i: (nb + i, 0)),
        ],
        out_specs=pl.BlockSpec((BLK, D), lambda i: (i, 0)),
        out_shape=jax.ShapeDtypeStruct((S, D), jnp.float32),
    )(z, z)


def _route(idx):
    e = idx.reshape(-1)
    oh = jax.nn.one_hot(e, E, dtype=jnp.int32)
    counts = oh.sum(axis=0)
    rank = (jnp.cumsum(oh, axis=0) - oh)[jnp.arange(S * K), e]
    blocks_per_e = (counts + BLK - 1) // BLK
    blk_start_e = jnp.cumsum(blocks_per_e) - blocks_per_e
    pos = blk_start_e[e] * BLK + rank
    nact = jnp.sum(blocks_per_e)
    cumblocks = jnp.cumsum(blocks_per_e)
    bids = jnp.arange(NB, dtype=jnp.int32)
    blk_expert = jnp.minimum(
        jnp.searchsorted(cumblocks, bids, side="right").astype(jnp.int32), E - 1)
    sorted_token = jnp.zeros((NP,), jnp.int32).at[pos].set(
        jnp.arange(S * K, dtype=jnp.int32) // K)
    return (pos.reshape(S, K), blk_expert, nact.reshape(1).astype(jnp.int32),
            sorted_token)


def kernel(x, Wg, bg, W1, b1, W2, b2):
    Bx, Sx, Dx = x.shape
    x_flat = x.reshape(-1, Dx)
    logits, idx, w = _gating(x_flat, Wg, bg)
    pos, blk_expert, nact, sorted_token = _route(idx)
    sorted_wt = jnp.zeros((NP,), jnp.float32).at[pos.reshape(-1)].set(w.reshape(-1))
    xs = _sc_gather(x_flat, sorted_token, NP)
    ys = _ffn(xs, W1, b1, W2, b2, sorted_wt, blk_expert, nact)
    z = _sc_gather(ys, jnp.concatenate([pos[:, 0], pos[:, 1]]), 2 * S)
    y = _tc_add(z)
    return (y.reshape(Bx, Sx, Dx), logits.reshape(Bx, Sx, E),
            idx.reshape(Bx, Sx, K))


# speedup vs baseline: 1.4887x; 1.4753x over previous
import functools
import numpy as np
import jax, jax.numpy as jnp
from jax import lax
from jax.experimental import pallas as pl
from jax.experimental.pallas import tpu as pltpu
from jax.experimental.pallas import tpu_sc as plsc

S, D, H, E, K = 2048, 1024, 2048, 8, 2
BLK = 256
NB = (S * K) // BLK + E
NP = NB * BLK


def _gating_body(x_ref, wg_ref, bg_ref, logits_ref, idx_ref, w_ref):
    x = x_ref[...]
    lg = jax.lax.dot_general(x, wg_ref[...], (((1,), (0,)), ((), ())),
                             preferred_element_type=jnp.float32,
                             precision=jax.lax.Precision.DEFAULT)
    lg = lg + bg_ref[...]
    logits_ref[...] = lg
    ii = jax.lax.broadcasted_iota(jnp.int32, lg.shape, 1)
    m0 = jnp.max(lg, axis=1, keepdims=True)
    i0 = jnp.min(jnp.where(lg == m0, ii, E), axis=1, keepdims=True)
    lg2 = jnp.where(ii == i0, -jnp.inf, lg)
    m1 = jnp.max(lg2, axis=1, keepdims=True)
    i1 = jnp.min(jnp.where(lg2 == m1, ii, E), axis=1, keepdims=True)
    t = jnp.exp(m1 - m0)
    w0 = 1.0 / (1.0 + t)
    w1 = t / (1.0 + t)
    idx_ref[...] = jnp.concatenate([i0, i1], axis=1)
    w_ref[...] = jnp.concatenate([w0, w1], axis=1)


def _gating(x_flat, Wg, bg):
    return pl.pallas_call(
        _gating_body,
        out_shape=(
            jax.ShapeDtypeStruct((S, E), jnp.float32),
            jax.ShapeDtypeStruct((S, K), jnp.int32),
            jax.ShapeDtypeStruct((S, K), jnp.float32),
        ),
    )(x_flat, Wg, bg.reshape(1, E))


def _ffn_body(be_ref, nact_ref, xs_ref, w1_ref, b1_ref, w2_ref, b2_ref, wt_ref,
              out_ref):
    @pl.when(pl.program_id(0) < nact_ref[0])
    def _():
        h = jax.lax.dot_general(xs_ref[...], w1_ref[0], (((1,), (0,)), ((), ())),
                                preferred_element_type=jnp.float32,
                                precision=jax.lax.Precision.DEFAULT)
        h = h + b1_ref[0]
        h = 0.5 * h * (1.0 + jax.lax.erf(h / np.sqrt(2).astype(np.float32)))
        o = jax.lax.dot_general(h, w2_ref[0], (((1,), (0,)), ((), ())),
                                preferred_element_type=jnp.float32,
                                precision=jax.lax.Precision.DEFAULT)
        out_ref[...] = (o + b2_ref[0]) * wt_ref[...]


def _ffn(xs, W1, b1, W2, b2, sorted_wt, blk_expert, nact):
    grid_spec = pltpu.PrefetchScalarGridSpec(
        num_scalar_prefetch=2,
        grid=(NB,),
        in_specs=[
            pl.BlockSpec((BLK, D), lambda i, be, na: (i, 0)),
            pl.BlockSpec((1, D, H), lambda i, be, na: (be[i], 0, 0)),
            pl.BlockSpec((1, 1, H), lambda i, be, na: (be[i], 0, 0)),
            pl.BlockSpec((1, H, D), lambda i, be, na: (be[i], 0, 0)),
            pl.BlockSpec((1, 1, D), lambda i, be, na: (be[i], 0, 0)),
            pl.BlockSpec((BLK, 1), lambda i, be, na: (i, 0)),
        ],
        out_specs=pl.BlockSpec((BLK, D), lambda i, be, na: (i, 0)),
    )
    return pl.pallas_call(
        _ffn_body,
        grid_spec=grid_spec,
        out_shape=jax.ShapeDtypeStruct((NP, D), jnp.float32),
        compiler_params=pltpu.CompilerParams(
            dimension_semantics=("arbitrary",)),
    )(blk_expert, nact, xs, W1, b1.reshape(E, 1, H), W2, b2.reshape(E, 1, D),
      sorted_wt.reshape(NP, 1))


NW = 32
GCHUNK = 32


def _sc_gather(table, idx, n_rows):
    per_w = n_rows // NW
    nchunk = per_w // GCHUNK
    mesh = plsc.VectorSubcoreMesh(core_axis_name="c", subcore_axis_name="s")

    @functools.partial(
        pl.kernel, mesh=mesh,
        out_type=jax.ShapeDtypeStruct((n_rows, D), jnp.float32),
        scratch_types=[
            pltpu.VMEM((GCHUNK,), jnp.int32),
            pltpu.VMEM((GCHUNK,), jnp.int32),
            pltpu.VMEM((GCHUNK, D), jnp.float32),
            pltpu.VMEM((GCHUNK, D), jnp.float32),
            pltpu.SemaphoreType.DMA,
            pltpu.SemaphoreType.DMA,
        ],
    )
    def k(tab_hbm, idx_hbm, out_hbm, i0_v, i1_v, r0_v, r1_v, sem0, sem1):
        wid = lax.axis_index("s") * 2 + lax.axis_index("c")
        ivs, rvs, sems = (i0_v, i1_v), (r0_v, r1_v), (sem0, sem1)

        def base(c):
            return wid * per_w + c * GCHUNK

        pltpu.sync_copy(idx_hbm.at[pl.ds(base(0), GCHUNK)], i0_v)
        cps = {0: pltpu.async_copy(tab_hbm.at[i0_v], r0_v, sem0)}
        for c in range(nchunk):
            b, nb = c % 2, (c + 1) % 2
            if c + 1 < nchunk:
                pltpu.sync_copy(idx_hbm.at[pl.ds(base(c + 1), GCHUNK)], ivs[nb])
            cps[c].wait()
            if c + 1 < nchunk:
                cps[c + 1] = pltpu.async_copy(tab_hbm.at[ivs[nb]], rvs[nb],
                                              sems[nb])
            pltpu.sync_copy(rvs[b], out_hbm.at[pl.ds(base(c), GCHUNK)])

    return k(table, idx)


def _add_body(a_ref, b_ref, y_ref):
    y_ref[...] = a_ref[...] + b_ref[...]


def _tc_add(z):
    nb = S // BLK
    return pl.pallas_call(
        _add_body,
        grid=(nb,),
        in_specs=[
            pl.BlockSpec((BLK, D), lambda i: (i, 0)),
            pl.BlockSpec((BLK, D), lambda i: (nb + i, 0)),
        ],
        out_specs=pl.BlockSpec((BLK, D), lambda i: (i, 0)),
        out_shape=jax.ShapeDtypeStruct((S, D), jnp.float32),
    )(z, z)


def _route(idx):
    e = idx.reshape(-1)
    oh = jax.nn.one_hot(e, E, dtype=jnp.int32)
    counts = oh.sum(axis=0)
    rank = (jnp.cumsum(oh, axis=0) - oh)[jnp.arange(S * K), e]
    blocks_per_e = (counts + BLK - 1) // BLK
    blk_start_e = jnp.cumsum(blocks_per_e) - blocks_per_e
    pos = blk_start_e[e] * BLK + rank
    nact = jnp.sum(blocks_per_e)
    cumblocks = jnp.cumsum(blocks_per_e)
    bids = jnp.arange(NB, dtype=jnp.int32)
    blk_expert = jnp.minimum(
        jnp.searchsorted(cumblocks, bids, side="right").astype(jnp.int32), E - 1)
    sorted_token = (jnp.arange(NP, dtype=jnp.int32) % S).at[pos].set(
        jnp.arange(S * K, dtype=jnp.int32) // K)
    return (pos.reshape(S, K), blk_expert, nact.reshape(1).astype(jnp.int32),
            sorted_token)


def kernel(x, Wg, bg, W1, b1, W2, b2):
    Bx, Sx, Dx = x.shape
    x_flat = x.reshape(-1, Dx)
    logits, idx, w = _gating(x_flat, Wg, bg)
    pos, blk_expert, nact, sorted_token = _route(idx)
    sorted_wt = jnp.zeros((NP,), jnp.float32).at[pos.reshape(-1)].set(w.reshape(-1))
    xs = _sc_gather(x_flat, sorted_token, NP)
    ys = _ffn(xs, W1, b1, W2, b2, sorted_wt, blk_expert, nact)
    z = _sc_gather(ys, jnp.concatenate([pos[:, 0], pos[:, 1]]), 2 * S)
    y = _tc_add(z)
    return (y.reshape(Bx, Sx, Dx), logits.reshape(Bx, Sx, E),
            idx.reshape(Bx, Sx, K))
